# Initial kernel scaffold; baseline (speedup 1.0000x reference)
import jax
import jax.numpy as jnp
from jax.experimental import pallas as pl


def kernel(hidden_states, Wr, Wg, Wu, Wd):
    raise NotImplementedError("write your pallas kernel here")


# speedup vs baseline: 2.0100x; 2.0100x over previous
import functools

import jax
import jax.numpy as jnp
from jax import lax
from jax.experimental import pallas as pl
from jax.experimental.pallas import tpu as pltpu
from jax.experimental.pallas import tpu_sc as plsc

EPS = 1e-5
NE = 8
D = 768
F = 3072
T = 2048
NC = 3
FC = F // NC
SLOTS = 2 * T
TB = 256
PADT = SLOTS + NE * TB
NBLK = PADT // TB
NWORK = 32
SPW = SLOTS // NWORK


def _scales_body(wg_ref, wu_ref, wd_ref, sg_ref, su_ref, sd_ref):
    j = pl.program_id(1)

    @pl.when(j == 0)
    def _():
        sg_ref[...] = jnp.zeros_like(sg_ref)
        su_ref[...] = jnp.zeros_like(su_ref)
        sd_ref[...] = jnp.zeros_like(sd_ref)

    sg_ref[...] += jnp.sum(jnp.abs(wg_ref[...]))
    su_ref[...] += jnp.sum(jnp.abs(wu_ref[...]))
    sd_ref[...] += jnp.sum(jnp.abs(wd_ref[...]))

    @pl.when(j == NC - 1)
    def _():
        n = float(F * D)
        sg_ref[...] = 1.0 / jnp.clip(sg_ref[...] / n, EPS, None)
        su_ref[...] = 1.0 / jnp.clip(su_ref[...] / n, EPS, None)
        sd_ref[...] = 1.0 / jnp.clip(sd_ref[...] / n, EPS, None)


def _compute_scales(Wg, Wu, Wd):
    return pl.pallas_call(
        _scales_body,
        grid=(NE, NC),
        in_specs=[
            pl.BlockSpec((1, FC, D), lambda e, j: (e, j, 0)),
            pl.BlockSpec((1, FC, D), lambda e, j: (e, j, 0)),
            pl.BlockSpec((1, D, FC), lambda e, j: (e, 0, j)),
        ],
        out_specs=[
            pl.BlockSpec((1, 1, 1), lambda e, j: (e, 0, 0)),
            pl.BlockSpec((1, 1, 1), lambda e, j: (e, 0, 0)),
            pl.BlockSpec((1, 1, 1), lambda e, j: (e, 0, 0)),
        ],
        out_shape=[jax.ShapeDtypeStruct((NE, 1, 1), jnp.float32)] * 3,
    )(Wg, Wu, Wd)


def _quant_body(wg_ref, wu_ref, wd_ref, sg_ref, su_ref, sd_ref,
                tg_ref, tu_ref, td_ref):
    def q(w, s):
        return (jnp.clip(jnp.round(w * s), -1.0, 1.0) / s).astype(jnp.bfloat16)

    tg_ref[...] = q(wg_ref[...], sg_ref[...])
    tu_ref[...] = q(wu_ref[...], su_ref[...])
    td_ref[...] = q(wd_ref[...], sd_ref[...])


def _quantize_weights(Wg, Wu, Wd, sg, su, sd):
    return pl.pallas_call(
        _quant_body,
        grid=(NE, NC),
        in_specs=[
            pl.BlockSpec((1, FC, D), lambda e, j: (e, j, 0)),
            pl.BlockSpec((1, FC, D), lambda e, j: (e, j, 0)),
            pl.BlockSpec((1, D, FC), lambda e, j: (e, 0, j)),
            pl.BlockSpec((1, 1, 1), lambda e, j: (e, 0, 0)),
            pl.BlockSpec((1, 1, 1), lambda e, j: (e, 0, 0)),
            pl.BlockSpec((1, 1, 1), lambda e, j: (e, 0, 0)),
        ],
        out_specs=[
            pl.BlockSpec((1, FC, D), lambda e, j: (e, j, 0)),
            pl.BlockSpec((1, FC, D), lambda e, j: (e, j, 0)),
            pl.BlockSpec((1, D, FC), lambda e, j: (e, 0, j)),
        ],
        out_shape=[
            jax.ShapeDtypeStruct((NE, F, D), jnp.bfloat16),
            jax.ShapeDtypeStruct((NE, F, D), jnp.bfloat16),
            jax.ShapeDtypeStruct((NE, D, F), jnp.bfloat16),
        ],
    )(Wg, Wu, Wd, sg, su, sd)


def _lane_prefix(x):
    n = x.shape[1]
    k = 1
    while k < n:
        shifted = jnp.concatenate(
            [jnp.zeros((x.shape[0], k), x.dtype), x[:, :n - k]], axis=1)
        x = x + shifted
        k *= 2
    return x


def _sublane_prefix(x):
    n = x.shape[0]
    k = 1
    while k < n:
        shifted = jnp.concatenate(
            [jnp.zeros((k, x.shape[1]), x.dtype), x[:n - k, :]], axis=0)
        x = x + shifted
        k *= 2
    return x


def _router_body(x_ref, wr_ref, w0_ref, w1_ref, pos_ref, bexp_ref):
    xb = x_ref[...].astype(jnp.bfloat16)
    logits = jax.lax.dot_general(
        xb, wr_ref[...].astype(jnp.bfloat16), (((1,), (1,)), ((), ())),
        preferred_element_type=jnp.float32)
    m = jnp.max(logits, axis=-1, keepdims=True)
    p = jnp.exp(logits - m)
    p = p / jnp.sum(p, axis=-1, keepdims=True)
    cnt = jnp.zeros_like(p)
    eidx = jax.lax.broadcasted_iota(jnp.int32, p.shape, 1)
    for ep in range(NE):
        col = p[:, ep:ep + 1]
        cnt = cnt + (col > p).astype(jnp.float32)
        cnt = cnt + ((col == p) & (eidx > ep)).astype(jnp.float32)
    is0 = cnt == 0.0
    is1 = cnt == 1.0
    v0 = jnp.sum(jnp.where(is0, p, 0.0), axis=1, keepdims=True)
    v1 = jnp.sum(jnp.where(is1, p, 0.0), axis=1, keepdims=True)
    denom = v0 + v1
    w0_ref[...] = v0 / denom
    w1_ref[...] = v1 / denom
    i0 = jnp.sum(jnp.where(is0, eidx, 0), axis=1, keepdims=True)
    i1 = jnp.sum(jnp.where(is1, eidx, 0), axis=1, keepdims=True)

    km = jnp.concatenate(
        [i0.reshape(T // 128, 128), i1.reshape(T // 128, 128)], axis=0)

    pos = jnp.zeros((SLOTS // 128, 128), jnp.int32)
    bexp = jnp.zeros((1, 128), jnp.int32)
    biota = jax.lax.broadcasted_iota(jnp.int32, (1, 128), 1)
    start = jnp.int32(0)
    startb = jnp.int32(0)
    for e in range(NE):
        me = (km == e).astype(jnp.int32)
        pfx = _lane_prefix(me)
        rowsum = pfx[:, 127:128]
        rpfx = _sublane_prefix(rowsum)
        rank0 = pfx + (rpfx - rowsum) - 1
        cnt_e = jnp.sum(me)
        pos = jnp.where(me == 1, start + rank0, pos)
        nb = (cnt_e + TB - 1) // TB
        endb = startb + nb
        bexp = bexp + (biota >= endb).astype(jnp.int32)
        start = start + nb * TB
        startb = endb
    pos_ref[...] = pos
    bexp_ref[...] = jnp.minimum(bexp, NE - 1)


def _route(x, Wr):
    return pl.pallas_call(
        _router_body,
        out_shape=[
            jax.ShapeDtypeStruct((T, 1), jnp.float32),
            jax.ShapeDtypeStruct((T, 1), jnp.float32),
            jax.ShapeDtypeStruct((SLOTS // 128, 128), jnp.int32),
            jax.ShapeDtypeStruct((1, 128), jnp.int32),
        ],
    )(x, Wr)


def _sc_dispatch(x, tok, pos):
    mesh = plsc.VectorSubcoreMesh(core_axis_name="c", subcore_axis_name="s")

    @functools.partial(
        pl.kernel, mesh=mesh,
        out_type=jax.ShapeDtypeStruct((PADT, D), jnp.float32),
        scratch_types=[
            pltpu.VMEM((SPW,), jnp.int32),
            pltpu.VMEM((SPW,), jnp.int32),
            pltpu.VMEM((SPW, D), jnp.float32),
            pltpu.SemaphoreType.DMA,
        ],
    )
    def k(x_hbm, tok_hbm, pos_hbm, xg_hbm, tok_v, pos_v, rows_v, sem):
        wid = lax.axis_index("s") * 2 + lax.axis_index("c")
        base = wid * SPW
        pltpu.sync_copy(tok_hbm.at[pl.ds(base, SPW)], tok_v)
        pltpu.sync_copy(pos_hbm.at[pl.ds(base, SPW)], pos_v)
        pltpu.async_copy(x_hbm.at[tok_v], rows_v, sem).wait()
        pltpu.async_copy(rows_v, xg_hbm.at[pos_v], sem).wait()

    return k(x, tok, pos)


def _sc_collect(y, pos):
    mesh = plsc.VectorSubcoreMesh(core_axis_name="c", subcore_axis_name="s")

    @functools.partial(
        pl.kernel, mesh=mesh,
        out_type=jax.ShapeDtypeStruct((SLOTS, D), jnp.float32),
        scratch_types=[
            pltpu.VMEM((SPW,), jnp.int32),
            pltpu.VMEM((SPW, D), jnp.float32),
            pltpu.SemaphoreType.DMA,
        ],
    )
    def k(y_hbm, pos_hbm, g_hbm, pos_v, rows_v, sem):
        wid = lax.axis_index("s") * 2 + lax.axis_index("c")
        base = wid * SPW
        pltpu.sync_copy(pos_hbm.at[pl.ds(base, SPW)], pos_v)
        pltpu.async_copy(y_hbm.at[pos_v], rows_v, sem).wait()
        pltpu.sync_copy(rows_v, g_hbm.at[pl.ds(base, SPW)])

    return k(y, pos)


def _moe_body(bexp_ref, xg_ref, tg_ref, tu_ref, td_ref, out_ref):
    x = xg_ref[...]
    xmax = jnp.clip(jnp.max(jnp.abs(x), axis=-1, keepdims=True), EPS, None)
    xs = 127.0 / xmax
    xq = (jnp.round(x * xs) / xs).astype(jnp.bfloat16)

    gate = jax.lax.dot_general(
        xq, tg_ref[0], (((1,), (1,)), ((), ())),
        preferred_element_type=jnp.float32)
    up = jax.lax.dot_general(
        xq, tu_ref[0], (((1,), (1,)), ((), ())),
        preferred_element_type=jnp.float32)
    act = up * jnp.maximum(gate, 0.0) ** 2

    amax = jnp.clip(jnp.max(jnp.abs(act), axis=-1, keepdims=True), EPS, None)
    as_ = 127.0 / amax
    aq = (jnp.round(act * as_) / as_).astype(jnp.bfloat16)
    out_ref[...] = jax.lax.dot_general(
        aq, td_ref[0], (((1,), (1,)), ((), ())),
        preferred_element_type=jnp.float32)


def _moe_blocked(xg, Tg, Tu, Td, bexp):
    grid_spec = pltpu.PrefetchScalarGridSpec(
        num_scalar_prefetch=1,
        grid=(NBLK,),
        in_specs=[
            pl.BlockSpec((TB, D), lambda b, be: (b, 0)),
            pl.BlockSpec((1, F, D), lambda b, be: (be[b], 0, 0)),
            pl.BlockSpec((1, F, D), lambda b, be: (be[b], 0, 0)),
            pl.BlockSpec((1, D, F), lambda b, be: (be[b], 0, 0)),
        ],
        out_specs=pl.BlockSpec((TB, D), lambda b, be: (b, 0)),
    )
    return pl.pallas_call(
        _moe_body,
        grid_spec=grid_spec,
        out_shape=jax.ShapeDtypeStruct((PADT, D), jnp.float32),
    )(bexp, xg, Tg, Tu, Td)


def _combine_body(g0_ref, g1_ref, w0_ref, w1_ref, out_ref):
    out_ref[...] = g0_ref[...] * w0_ref[...] + g1_ref[...] * w1_ref[...]


def _combine(g, w0, w1, tb=256):
    nt = T // tb
    return pl.pallas_call(
        _combine_body,
        grid=(nt,),
        in_specs=[
            pl.BlockSpec((tb, D), lambda t: (t, 0)),
            pl.BlockSpec((tb, D), lambda t, _n=T // tb: (t + _n, 0)),
            pl.BlockSpec((tb, 1), lambda t: (t, 0)),
            pl.BlockSpec((tb, 1), lambda t: (t, 0)),
        ],
        out_specs=pl.BlockSpec((tb, D), lambda t: (t, 0)),
        out_shape=jax.ShapeDtypeStruct((T, D), jnp.float32),
    )(g, g, w0, w1)


def kernel(hidden_states, Wr, Wg, Wu, Wd):
    B, S, Dh = hidden_states.shape
    x = hidden_states.reshape(-1, Dh)
    w0, w1, pos2d, bexp2d = _route(x, Wr)
    pos = pos2d.reshape(-1)
    bexp = bexp2d.reshape(-1)[:NBLK]
    tok = jnp.concatenate([jnp.arange(T, dtype=jnp.int32)] * 2)
    xg = _sc_dispatch(x, tok, pos)
    sg, su, sd = _compute_scales(Wg, Wu, Wd)
    Tg, Tu, Td = _quantize_weights(Wg, Wu, Wd, sg, su, sd)
    y = _moe_blocked(xg, Tg, Tu, Td, bexp)
    g = _sc_collect(y, pos)
    out = _combine(g, w0, w1)
    return out.reshape(B, S, Dh)
